# baseline (speedup 1.0000x reference)
import jax
import jax.numpy as jnp
from jax import lax
from jax.experimental import pallas as pl
from jax.experimental.pallas import tpu as pltpu
from jax.experimental.pallas import tpu_sc as plsc

_NC = 2
_NS = 16
_NW = _NC * _NS
_CHUNK = 32
_NBUF = 5


def _make_sc_gather(n_idx, d_model, dtype):
    per_w = n_idx // _NW
    n_chunks = per_w // _CHUNK
    mesh = plsc.VectorSubcoreMesh(core_axis_name="c", subcore_axis_name="s")

    def body(idx_hbm, table_hbm, out_hbm, idx_v, *bufs_and_sems):
        rows = bufs_and_sems[:_NBUF]
        gsems = bufs_and_sems[_NBUF:2 * _NBUF]
        ssems = bufs_and_sems[2 * _NBUF:]
        wid = lax.axis_index("s") * _NC + lax.axis_index("c")
        base = wid * per_w
        pltpu.sync_copy(idx_hbm.at[wid], idx_v)

        def gather(j):
            b = j % _NBUF
            return pltpu.async_copy(table_hbm.at[idx_v.at[j]], rows[b], gsems[b])

        gets = [None] * n_chunks
        puts = [None] * n_chunks
        put_waited = [False] * n_chunks
        for j in range(min(_NBUF, n_chunks)):
            gets[j] = gather(j)
        for j in range(n_chunks):
            b = j % _NBUF
            if j >= 3 and j + 2 < n_chunks:
                puts[j - 3].wait()
                put_waited[j - 3] = True
                gets[j + 2] = gather(j + 2)
            gets[j].wait()
            puts[j] = pltpu.async_copy(
                rows[b], out_hbm.at[pl.ds(base + j * _CHUNK, _CHUNK)], ssems[b])
        for j in range(n_chunks):
            if not put_waited[j]:
                puts[j].wait()

    return pl.kernel(
        body,
        mesh=mesh,
        out_type=jax.ShapeDtypeStruct((n_idx, d_model), dtype),
        scratch_types=[
            pltpu.VMEM((n_chunks, _CHUNK), jnp.int32),
            *[pltpu.VMEM((_CHUNK, d_model), dtype) for _ in range(_NBUF)],
            *[pltpu.SemaphoreType.DMA for _ in range(2 * _NBUF)],
        ],
    )


def _make_tc_sine(n_idx, d_model, dtype, rows_per_blk=512):
    import math
    import numpy as np
    n_blk = n_idx // rows_per_blk

    def body(pos_ref, omega_ref, phase_ref, out_ref):
        pos = pos_ref[0, 0, :].astype(jnp.float32)
        angle = pos[:, None] * omega_ref[0, :][None, :] + phase_ref[0, :][None, :]
        out_ref[...] = jnp.sin(angle)

    return pl.pallas_call(
        body,
        grid=(n_blk,),
        in_specs=[
            pl.BlockSpec((1, 1, rows_per_blk), lambda i: (i, 0, 0)),
            pl.BlockSpec((1, d_model), lambda i: (0, 0)),
            pl.BlockSpec((1, d_model), lambda i: (0, 0)),
        ],
        out_specs=pl.BlockSpec((rows_per_blk, d_model), lambda i: (i, 0)),
        out_shape=jax.ShapeDtypeStruct((n_idx, d_model), dtype),
    )


def _sine_consts(d_model):
    import math
    import numpy as np
    div_term = np.exp(np.arange(0, d_model, 2, dtype=np.float32)
                      * (-math.log(10000.0) / d_model))
    omega = np.repeat(div_term, 2).reshape(1, d_model)
    phase = np.tile(np.array([0.0, math.pi / 2], dtype=np.float32),
                    d_model // 2).reshape(1, d_model)
    return jnp.asarray(omega), jnp.asarray(phase)


def kernel(position_ids, pe):
    b, s = position_ids.shape
    _, d = pe.shape
    n = b * s
    n_tc = n // 4
    n_sc = n - n_tc
    flat = position_ids.reshape(n)
    per_w = n_sc // _NW
    idx_sc = flat[:n_sc].reshape(_NW, per_w // _CHUNK, _CHUNK)
    rpb = 512
    pos3 = flat[n_sc:].reshape(n_tc // rpb, 1, rpb)
    omega, phase = _sine_consts(d)
    out_sc = _make_sc_gather(n_sc, d, pe.dtype)(idx_sc, pe)
    angle = (flat[n_sc:, None].astype(jnp.float32) * omega[0][None, :]
             + phase[0][None, :])
    out_tc = jnp.sin(angle)
    out = jnp.concatenate([out_sc, out_tc], axis=0)
    return out.reshape(b, s, d)


def _kernel_sc_only(position_ids, pe):
    b, s = position_ids.shape
    _, d = pe.shape
    n = b * s
    per_w = n // _NW
    idx = position_ids.reshape(_NW, per_w // _CHUNK, _CHUNK)
    out = _make_sc_gather(n, d, pe.dtype)(idx, pe)
    return out.reshape(b, s, d)


# speedup vs baseline: 1.8060x; 1.8060x over previous
import jax
import jax.numpy as jnp
from jax import lax
from jax.experimental import pallas as pl
from jax.experimental.pallas import tpu as pltpu
from jax.experimental.pallas import tpu_sc as plsc

_NC = 2
_NS = 16
_NW = _NC * _NS
_CHUNK = 32
_NBUF = 5


def _make_sc_gather(n_idx, d_model, dtype):
    per_w = n_idx // _NW
    n_chunks = per_w // _CHUNK
    mesh = plsc.VectorSubcoreMesh(core_axis_name="c", subcore_axis_name="s")

    def body(idx_hbm, table_hbm, out_hbm, idx_v, *bufs_and_sems):
        rows = bufs_and_sems[:_NBUF]
        gsems = bufs_and_sems[_NBUF:2 * _NBUF]
        ssems = bufs_and_sems[2 * _NBUF:]
        wid = lax.axis_index("s") * _NC + lax.axis_index("c")
        base = wid * per_w
        pltpu.sync_copy(idx_hbm.at[wid], idx_v)

        def gather(j):
            b = j % _NBUF
            return pltpu.async_copy(table_hbm.at[idx_v.at[j]], rows[b], gsems[b])

        gets = [None] * n_chunks
        puts = [None] * n_chunks
        put_waited = [False] * n_chunks
        for j in range(min(_NBUF, n_chunks)):
            gets[j] = gather(j)
        for j in range(n_chunks):
            b = j % _NBUF
            if j >= 3 and j + 2 < n_chunks:
                puts[j - 3].wait()
                put_waited[j - 3] = True
                gets[j + 2] = gather(j + 2)
            gets[j].wait()
            puts[j] = pltpu.async_copy(
                rows[b], out_hbm.at[pl.ds(base + j * _CHUNK, _CHUNK)], ssems[b])
        for j in range(n_chunks):
            if not put_waited[j]:
                puts[j].wait()

    return pl.kernel(
        body,
        mesh=mesh,
        out_type=jax.ShapeDtypeStruct((n_idx, d_model), dtype),
        scratch_types=[
            pltpu.VMEM((n_chunks, _CHUNK), jnp.int32),
            *[pltpu.VMEM((_CHUNK, d_model), dtype) for _ in range(_NBUF)],
            *[pltpu.SemaphoreType.DMA for _ in range(2 * _NBUF)],
        ],
    )


def kernel(position_ids, pe):
    b, s = position_ids.shape
    _, d = pe.shape
    n = b * s
    per_w = n // _NW
    idx = position_ids.reshape(_NW, per_w // _CHUNK, _CHUNK)
    out = _make_sc_gather(n, d, pe.dtype)(idx, pe)
    return out.reshape(b, s, d)
